# baseline (speedup 1.0000x reference)
import functools

import jax
import jax.numpy as jnp
from jax import lax
from jax.experimental import pallas as pl
from jax.experimental.pallas import tpu as pltpu
from jax.experimental.pallas import tpu_sc as plsc

NC = 2
NS = 16
L = 16
NW = NC * NS

B = 1024
S = 200
D = 128
TOTAL = B * S
PER_W = TOTAL // NW
CHUNK = 128
NCHUNK = PER_W // CHUNK
VPC = CHUNK // L
SUP = 2 * CHUNK
NSUP = PER_W // SUP
NPAIR = 3


def _sc_gather(table, text_rows, seql):
    mesh = plsc.VectorSubcoreMesh(core_axis_name="c", subcore_axis_name="s")

    @functools.partial(
        pl.kernel,
        out_type=jax.ShapeDtypeStruct((TOTAL, D), jnp.float32),
        mesh=mesh,
        scratch_types=[
            pltpu.VMEM((NCHUNK, CHUNK), jnp.int32),
            pltpu.VMEM((16,), jnp.int32),
            pltpu.VMEM((NPAIR * SUP, D), jnp.float32),
            [pltpu.SemaphoreType.DMA for _ in range(NPAIR)],
            [pltpu.SemaphoreType.DMA for _ in range(NPAIR)],
        ],
    )
    def k(table_hbm, text_hbm, seql_hbm, out_hbm, idx_v, seql_v,
          ring, gsems, ssems):
        wid = lax.axis_index("s") * NC + lax.axis_index("c")
        base = wid * PER_W
        pltpu.sync_copy(text_hbm.at[wid], idx_v)
        pltpu.sync_copy(seql_hbm, seql_v)
        seql = seql_v[...]
        lane = lax.iota(jnp.int32, L)

        @pl.loop(0, NCHUNK)
        def _fix(r):
            for v in range(VPC):
                off = r * CHUNK + v * L
                vec = idx_v[r, pl.ds(v * L, L)]
                pos = lax.rem(off + lane, S)
                idx_v[r, pl.ds(v * L, L)] = jnp.where(pos < seql, vec + 1, 0)

        def start_gathers(g, p):
            pltpu.async_copy(table_hbm.at[idx_v.at[2 * g]],
                             ring.at[pl.ds(p * SUP, CHUNK)], gsems[p])
            pltpu.async_copy(table_hbm.at[idx_v.at[2 * g + 1]],
                             ring.at[pl.ds(p * SUP + CHUNK, CHUNK)], gsems[p])

        def wait_gathers(p):
            for h in range(2):
                pltpu.make_async_copy(
                    table_hbm.at[idx_v.at[0]],
                    ring.at[pl.ds(p * SUP + h * CHUNK, CHUNK)],
                    gsems[p]).wait()

        def start_store(g, p):
            pltpu.async_copy(ring.at[pl.ds(p * SUP, SUP)],
                             out_hbm.at[pl.ds(base + g * SUP, SUP)], ssems[p])

        def wait_store(p):
            pltpu.make_async_copy(ring.at[pl.ds(p * SUP, SUP)],
                                  out_hbm.at[pl.ds(base, SUP)], ssems[p]).wait()

        def step(g, p, last):
            wait_gathers(p)
            start_store(g, p)
            if not last:
                nxtg = g + NPAIR - 1
                pn = (p + NPAIR - 1) % NPAIR

                @pl.when(nxtg < NSUP)
                def _():
                    @pl.when(g >= 1)
                    def _():
                        wait_store(pn)
                    start_gathers(nxtg, pn)

        for j in range(NPAIR - 1):
            start_gathers(j, j)

        @pl.loop(0, NSUP - 1, step=NPAIR)
        def _pipe(gbase):
            for u in range(NPAIR):
                step(gbase + u, u, last=False)

        step(NSUP - 1, (NSUP - 1) % NPAIR, last=True)

        for g in range(NSUP - NPAIR + 1, NSUP):
            wait_store(g % NPAIR)

    return k(table, text_rows, seql)


def kernel(lang, text, seq_len, table):
    del lang
    text_rows = text.astype(jnp.int32).reshape(NW, NCHUNK, CHUNK)
    seql = jnp.full((16,), seq_len, dtype=jnp.int32)
    out = _sc_gather(table, text_rows, seql)
    return out.reshape(B, S, D)


# speedup vs baseline: 1.0285x; 1.0285x over previous
import functools

import jax
import jax.numpy as jnp
from jax import lax
from jax.experimental import pallas as pl
from jax.experimental.pallas import tpu as pltpu
from jax.experimental.pallas import tpu_sc as plsc

NC = 2
NS = 16
L = 16
NW = NC * NS

B = 1024
S = 200
D = 128
TOTAL = B * S
PER_W = TOTAL // NW
CHUNK = 128
NCHUNK = PER_W // CHUNK
VPC = CHUNK // L
DEPTH = 7


def _sc_gather(table, text_rows, seql):
    mesh = plsc.VectorSubcoreMesh(core_axis_name="c", subcore_axis_name="s")

    @functools.partial(
        pl.kernel,
        out_type=jax.ShapeDtypeStruct((TOTAL, D), jnp.float32),
        mesh=mesh,
        scratch_types=[
            pltpu.VMEM((NCHUNK, CHUNK), jnp.int32),
            pltpu.VMEM((16,), jnp.int32),
            [pltpu.VMEM((CHUNK, D), jnp.float32) for _ in range(DEPTH)],
            [pltpu.SemaphoreType.DMA for _ in range(DEPTH)],
            [pltpu.SemaphoreType.DMA for _ in range(DEPTH)],
        ],
    )
    def k(table_hbm, text_hbm, seql_hbm, out_hbm, idx_v, seql_v,
          bufs, gsems, ssems):
        wid = lax.axis_index("s") * NC + lax.axis_index("c")
        base = wid * PER_W
        pltpu.sync_copy(text_hbm.at[wid], idx_v)
        pltpu.sync_copy(seql_hbm, seql_v)
        seql = seql_v[...]
        lane = lax.iota(jnp.int32, L)

        def fix(r):
            for v in range(VPC):
                off = r * CHUNK + v * L
                vec = idx_v[r, pl.ds(v * L, L)]
                pos = lax.rem(off + lane, S)
                idx_v[r, pl.ds(v * L, L)] = jnp.where(pos < seql, vec + 1, 0)

        def start_gather(c, b):
            pltpu.async_copy(table_hbm.at[idx_v.at[c]], bufs[b], gsems[b])

        def wait_gather(b):
            pltpu.make_async_copy(table_hbm.at[idx_v.at[0]], bufs[b],
                                  gsems[b]).wait()

        def start_store(c, b):
            pltpu.async_copy(bufs[b], out_hbm.at[pl.ds(base + c * CHUNK, CHUNK)],
                             ssems[b])

        def wait_store(b):
            pltpu.make_async_copy(bufs[b], out_hbm.at[pl.ds(base, CHUNK)],
                                  ssems[b]).wait()

        @pl.loop(0, NCHUNK)
        def _fix_all(r):
            fix(r)

        def step(c, b):
            wait_gather(b)
            start_store(c, b)
            nxt = c + DEPTH - 1
            bn = (b + DEPTH - 1) % DEPTH

            @pl.when(nxt < NCHUNK)
            def _():
                @pl.when(c >= 1)
                def _():
                    wait_store(bn)
                start_gather(nxt, bn)

        for j in range(DEPTH - 1):
            start_gather(j, j)

        @pl.loop(0, NCHUNK - 1, step=DEPTH)
        def _pipe(cbase):
            for b in range(DEPTH):
                step(cbase + b, b)

        step(NCHUNK - 1, (NCHUNK - 1) % DEPTH)

        for g in range(NCHUNK - DEPTH, NCHUNK):
            wait_store(g % DEPTH)

    return k(table, text_rows, seql)


def kernel(lang, text, seq_len, table):
    del lang
    text_rows = text.astype(jnp.int32).reshape(NW, NCHUNK, CHUNK)
    seql = jnp.full((16,), seq_len, dtype=jnp.int32)
    out = _sc_gather(table, text_rows, seql)
    return out.reshape(B, S, D)
